# baseline (speedup 1.0000x reference)
import jax
import jax.numpy as jnp
from jax import lax
from jax.experimental import pallas as pl
from jax.experimental.pallas import tpu as pltpu

EPS = 1e-5

_DN = ('NHWC', 'HWIO', 'NHWC')


def _hwio(w):
    return jnp.transpose(w, (2, 3, 1, 0))


def _conv(x, w, b, stride=(1, 1), padding=((0, 0), (0, 0))):
    out = lax.conv_general_dilated(x, _hwio(w), window_strides=stride,
                                   padding=padding, dimension_numbers=_DN)
    return out + b[None, None, None, :]


def _conv_t_b(x, w, b, stride, kernel, padding, out_pad):
    kh, kw = kernel
    ph, pw = padding
    oph, opw = out_pad
    pads = ((kh - 1 - ph, kh - 1 - ph + oph), (kw - 1 - pw, kw - 1 - pw + opw))
    out = lax.conv_general_dilated(x.astype(jnp.bfloat16),
                                   _hwio(w).astype(jnp.bfloat16),
                                   window_strides=(1, 1), padding=pads,
                                   lhs_dilation=stride, dimension_numbers=_DN,
                                   preferred_element_type=jnp.float32)
    return out + b[None, None, None, :]


def _bn(x, g, b):
    m = x.mean(axis=(0, 1, 2), keepdims=True)
    v = x.var(axis=(0, 1, 2), keepdims=True)
    return g[None, None, None, :] * (x - m) * lax.rsqrt(v + EPS) + b[None, None, None, :]


def _res(x, w1, b1, w2, b2):
    h = jax.nn.relu(x)
    h = _conv(h, w1, b1, (1, 1), ((1, 1), (1, 1)))
    h = jax.nn.relu(h)
    h = _conv(h, w2, b2)
    return x + h


_W = 56


def _res_in(x, wt9, b1r, w2t, b2r, scratch, mask_l, mask_r):
    S, D = x.shape
    halo = _W + 1
    r = jnp.maximum(x, 0.0)
    scratch[pl.ds(0, halo), :] = jnp.zeros((halo, D), jnp.float32)
    scratch[pl.ds(halo + S, halo), :] = jnp.zeros((halo, D), jnp.float32)
    scratch[pl.ds(halo, S), :] = r
    acc = None
    for ky in range(3):
        for kx in range(3):
            off = halo + (ky - 1) * _W + (kx - 1)
            src = scratch[pl.ds(off, S), :]
            if kx == 0:
                src = jnp.where(mask_l, src, 0.0)
            elif kx == 2:
                src = jnp.where(mask_r, src, 0.0)
            t = jnp.dot(src, wt9[3 * ky + kx],
                        preferred_element_type=jnp.float32)
            acc = t if acc is None else acc + t
    h = jnp.maximum(acc + b1r, 0.0)
    return x + jnp.dot(h, w2t, preferred_element_type=jnp.float32) + b2r


def _vq_body(h_ref, e_ref, et_ref,
             e3t_ref, e3b_ref, w1t_ref, b1_ref,
             q1wt_ref, q1b1_ref, q1w2t_ref, q1b2_ref,
             q2wt_ref, q2b1_ref, q2w2t_ref, q2b2_ref,
             w2t_ref, b2_ref, wpt_ref, bp_ref,
             r1wt_ref, r1b1_ref, r1w2t_ref, r1b2_ref,
             r2wt_ref, r2b1_ref, r2w2t_ref, r2b2_ref,
             pw2t_ref, pb2_ref,
             out_ref, loss_ref, scratch):
    h0 = h_ref[0]
    S = h0.shape[0]
    row = lax.broadcasted_iota(jnp.int32, (S, 1), 0) % _W
    mask_l = row != 0
    mask_r = row != (_W - 1)

    h = jnp.dot(h0, e3t_ref[:], preferred_element_type=jnp.float32) + e3b_ref[:]
    h = jnp.dot(h, w1t_ref[:], preferred_element_type=jnp.float32) + b1_ref[:]
    h = _res_in(h, q1wt_ref, q1b1_ref[:], q1w2t_ref[:], q1b2_ref[:],
                scratch, mask_l, mask_r)
    h = _res_in(h, q2wt_ref, q2b1_ref[:], q2w2t_ref[:], q2b2_ref[:],
                scratch, mask_l, mask_r)
    z = jnp.dot(h, w2t_ref[:], preferred_element_type=jnp.float32) + b2_ref[:]
    E = e_ref[:]
    e2 = jnp.sum(E * E, axis=0, keepdims=True)
    scores = e2 - 2.0 * jnp.dot(z, E, preferred_element_type=jnp.float32)
    idx = jnp.argmin(scores, axis=1)
    onehot = (lax.broadcasted_iota(jnp.int32, scores.shape, 1)
              == idx[:, None]).astype(jnp.float32)
    quant = jnp.dot(onehot, et_ref[:], preferred_element_type=jnp.float32)
    d = quant - z
    part = jnp.sum(d * d).reshape(1, 1)

    q = jnp.dot(quant, wpt_ref[:], preferred_element_type=jnp.float32) + bp_ref[:]
    q = _res_in(q, r1wt_ref, r1b1_ref[:], r1w2t_ref[:], r1b2_ref[:],
                scratch, mask_l, mask_r)
    q = _res_in(q, r2wt_ref, r2b1_ref[:], r2w2t_ref[:], r2b2_ref[:],
                scratch, mask_l, mask_r)
    out_ref[0] = jnp.dot(q, pw2t_ref[:],
                         preferred_element_type=jnp.float32) + pb2_ref[:]

    @pl.when(pl.program_id(0) == 0)
    def _():
        loss_ref[...] = jnp.zeros((1, 1), jnp.float32)

    loss_ref[...] += part


def _vq_pallas(h, E, p):
    B, S, D = h.shape
    K = E.shape[1]

    def mt(name):
        return p[name][:, :, 0, 0].T

    def rowb(name):
        return p[name][None, :]

    def taps(name):
        return jnp.transpose(p[name], (2, 3, 1, 0)).reshape(9, D, D)

    full = lambda shape: pl.BlockSpec(shape, lambda i: tuple(0 for _ in shape))
    operands = [
        E, E.T,
        mt('enc_w3'), rowb('enc_b3'), mt('pre_w1'), rowb('pre_b1'),
        taps('pre_r1_w1'), rowb('pre_r1_b1'), mt('pre_r1_w2'), rowb('pre_r1_b2'),
        taps('pre_r2_w1'), rowb('pre_r2_b1'), mt('pre_r2_w2'), rowb('pre_r2_b2'),
        mt('pre_w2'), rowb('pre_b2'), mt('post_w1'), rowb('post_b1'),
        taps('post_r1_w1'), rowb('post_r1_b1'), mt('post_r1_w2'), rowb('post_r1_b2'),
        taps('post_r2_w1'), rowb('post_r2_b1'), mt('post_r2_w2'), rowb('post_r2_b2'),
        mt('post_w2'), rowb('post_b2'),
    ]
    out, loss_sum = pl.pallas_call(
        _vq_body,
        grid=(B,),
        in_specs=[pl.BlockSpec((1, S, D), lambda i: (i, 0, 0))]
        + [full(op.shape) for op in operands],
        out_specs=[
            pl.BlockSpec((1, S, D), lambda i: (i, 0, 0)),
            pl.BlockSpec((1, 1), lambda i: (0, 0)),
        ],
        out_shape=[
            jax.ShapeDtypeStruct((B, S, D), jnp.float32),
            jax.ShapeDtypeStruct((1, 1), jnp.float32),
        ],
        scratch_shapes=[pltpu.VMEM((S + 2 * (_W + 1), D), jnp.float32)],
    )(h, *operands)
    return out, loss_sum[0, 0]


def kernel(x, params):
    p = params
    h = jnp.transpose(x, (0, 2, 3, 1))
    h = _conv(h, p['enc_w1'], p['enc_b1'], (2, 2), ((1, 1), (1, 1)))
    h = jax.nn.relu(_bn(h, p['enc_g1'], p['enc_be1']))
    h = _conv(h, p['enc_w2'], p['enc_b2'], (2, 2), ((1, 1), (1, 1)))
    h = jax.nn.relu(_bn(h, p['enc_g2'], p['enc_be2']))

    E = p['embedding']
    B, H, W, D = h.shape
    out_rm, loss_sum = _vq_pallas(h.reshape(B, H * W, D), E, p)
    loss = 1.25 * loss_sum / (B * D * H * W)
    h = out_rm.reshape(B, H, W, D)

    h = _conv_t_b(h, p['dec_w1'], p['dec_b1'], (2, 2), (4, 3), (1, 1), (0, 0))
    h = jax.nn.relu(_bn(h, p['dec_g1'], p['dec_be1']))
    recon = _conv_t_b(h, p['dec_w2'], p['dec_b2'], (2, 2), (4, 3), (1, 1), (0, 1))
    return jnp.transpose(recon, (0, 3, 1, 2)), loss


# speedup vs baseline: 1.2482x; 1.2482x over previous
import jax
import jax.numpy as jnp
from jax import lax
from jax.experimental import pallas as pl

EPS = 1e-5


def _conv(x, w, b, stride=(1, 1), padding=((0, 0), (0, 0))):
    out = lax.conv_general_dilated(x, w, window_strides=stride, padding=padding,
                                   dimension_numbers=('NCHW', 'OIHW', 'NCHW'))
    return out + b[None, :, None, None]


def _conv_b(x, w, b, stride=(1, 1), padding=((0, 0), (0, 0))):
    out = lax.conv_general_dilated(x.astype(jnp.bfloat16), w.astype(jnp.bfloat16),
                                   window_strides=stride, padding=padding,
                                   dimension_numbers=('NCHW', 'OIHW', 'NCHW'),
                                   preferred_element_type=jnp.float32)
    return out + b[None, :, None, None]


def _conv_t_b(x, w, b, stride, kernel, padding, out_pad):
    kh, kw = kernel
    ph, pw = padding
    oph, opw = out_pad
    pads = ((kh - 1 - ph, kh - 1 - ph + oph), (kw - 1 - pw, kw - 1 - pw + opw))
    out = lax.conv_general_dilated(x.astype(jnp.bfloat16), w.astype(jnp.bfloat16),
                                   window_strides=(1, 1), padding=pads,
                                   lhs_dilation=stride,
                                   dimension_numbers=('NCHW', 'OIHW', 'NCHW'),
                                   preferred_element_type=jnp.float32)
    return out + b[None, :, None, None]


def _bn(x, g, b):
    m = x.mean(axis=(0, 2, 3), keepdims=True)
    v = x.var(axis=(0, 2, 3), keepdims=True)
    return g[None, :, None, None] * (x - m) * lax.rsqrt(v + EPS) + b[None, :, None, None]


def _res(x, w1, b1, w2, b2):
    h = jax.nn.relu(x)
    h = _conv(h, w1, b1, (1, 1), ((1, 1), (1, 1)))
    h = jax.nn.relu(h)
    h = _conv(h, w2, b2)
    return x + h


def _res_b(x, w1, b1, w2, b2):
    h = jax.nn.relu(x)
    h = _conv_b(h, w1, b1, (1, 1), ((1, 1), (1, 1)))
    h = jax.nn.relu(h)
    h = _conv_b(h, w2, b2)
    return x + h


def _vq_body(h_ref, e_ref, et_ref, w2_ref, b2_ref, wp_ref, bp_ref,
             out_ref, loss_ref):
    h = h_ref[0]
    z = jnp.dot(w2_ref[:], h, preferred_element_type=jnp.float32) + b2_ref[:]
    et = et_ref[:]
    e2 = jnp.sum(et * et, axis=1, keepdims=True)
    scores = e2 - 2.0 * jnp.dot(et, z, preferred_element_type=jnp.float32)
    idx = jnp.argmin(scores, axis=0)
    onehot = (lax.broadcasted_iota(jnp.int32, scores.shape, 0)
              == idx[None, :]).astype(jnp.float32)
    quant = jnp.dot(e_ref[:], onehot, preferred_element_type=jnp.float32)
    d = quant - z
    part = jnp.sum(d * d).reshape(1, 1)
    out_ref[0] = jnp.dot(wp_ref[:], quant,
                         preferred_element_type=jnp.float32) + bp_ref[:]

    @pl.when(pl.program_id(0) == 0)
    def _():
        loss_ref[...] = jnp.zeros((1, 1), jnp.float32)

    loss_ref[...] += part


def _vq_pallas(h, E, w2, b2, wp, bp):
    B, D, S = h.shape
    K = E.shape[1]
    grid = (B,)
    out, loss_sum = pl.pallas_call(
        _vq_body,
        grid=grid,
        in_specs=[
            pl.BlockSpec((1, D, S), lambda i: (i, 0, 0)),
            pl.BlockSpec((D, K), lambda i: (0, 0)),
            pl.BlockSpec((K, D), lambda i: (0, 0)),
            pl.BlockSpec((D, D), lambda i: (0, 0)),
            pl.BlockSpec((D, 1), lambda i: (0, 0)),
            pl.BlockSpec((D, D), lambda i: (0, 0)),
            pl.BlockSpec((D, 1), lambda i: (0, 0)),
        ],
        out_specs=[
            pl.BlockSpec((1, D, S), lambda i: (i, 0, 0)),
            pl.BlockSpec((1, 1), lambda i: (0, 0)),
        ],
        out_shape=[
            jax.ShapeDtypeStruct((B, D, S), jnp.float32),
            jax.ShapeDtypeStruct((1, 1), jnp.float32),
        ],
    )(h, E, E.T, w2, b2, wp, bp)
    return out, loss_sum[0, 0]


def kernel(x, params):
    p = params
    h = _conv(x, p['enc_w1'], p['enc_b1'], (2, 2), ((1, 1), (1, 1)))
    h = jax.nn.relu(_bn(h, p['enc_g1'], p['enc_be1']))
    h = _conv(h, p['enc_w2'], p['enc_b2'], (2, 2), ((1, 1), (1, 1)))
    h = jax.nn.relu(_bn(h, p['enc_g2'], p['enc_be2']))
    h = _conv(h, p['enc_w3'], p['enc_b3'])
    h = _conv(h, p['pre_w1'], p['pre_b1'])
    h = _res(h, p['pre_r1_w1'], p['pre_r1_b1'], p['pre_r1_w2'], p['pre_r1_b2'])
    h = _res(h, p['pre_r2_w1'], p['pre_r2_b1'], p['pre_r2_w2'], p['pre_r2_b2'])

    E = p['embedding']
    B, D, H, W = h.shape
    out, loss_sum = _vq_pallas(
        h.reshape(B, D, H * W), E,
        p['pre_w2'][:, :, 0, 0], p['pre_b2'][:, None],
        p['post_w1'][:, :, 0, 0], p['post_b1'][:, None])
    loss = 1.25 * loss_sum / (B * D * H * W)
    h = out.reshape(B, D, H, W)

    h = _res_b(h, p['post_r1_w1'], p['post_r1_b1'], p['post_r1_w2'], p['post_r1_b2'])
    h = _res_b(h, p['post_r2_w1'], p['post_r2_b1'], p['post_r2_w2'], p['post_r2_b2'])
    h = _conv_b(h, p['post_w2'], p['post_b2'])
    h = _conv_t_b(h, p['dec_w1'], p['dec_b1'], (2, 2), (4, 3), (1, 1), (0, 0))
    h = jax.nn.relu(_bn(h, p['dec_g1'], p['dec_be1']))
    recon = _conv_t_b(h, p['dec_w2'], p['dec_b2'], (2, 2), (4, 3), (1, 1), (0, 1))
    return recon, loss
